# Initial kernel scaffold; baseline (speedup 1.0000x reference)
import jax
import jax.numpy as jnp
from jax.experimental import pallas as pl


def kernel(logits, labels):
    raise NotImplementedError("write your pallas kernel here")


# speedup vs baseline: 80.7295x; 80.7295x over previous
import functools

import jax
import jax.numpy as jnp
from jax import lax
from jax.experimental import pallas as pl
from jax.experimental.pallas import tpu as pltpu
from jax.experimental.pallas import tpu_sc as plsc

ROWS = 1024
COLS = 100000
LANES = 16
NUM_CORES = 2
NUM_SUBCORES = 16
NUM_WORKERS = NUM_CORES * NUM_SUBCORES
ROWS_PER_WORKER = ROWS // NUM_WORKERS
NVREG = COLS // LANES
KAPPA = jnp.float32(1e30)
NEG_INF = jnp.float32(-jnp.inf)


_GATHER_DNUMS = lax.GatherDimensionNumbers(
    offset_dims=(), collapsed_slice_dims=(0,), start_index_map=(0,)
)


def _shuffle(v, idx):
    return lax.gather(
        v,
        idx.reshape(LANES, 1),
        _GATHER_DNUMS,
        slice_sizes=(1,),
        mode=lax.GatherScatterMode.PROMISE_IN_BOUNDS,
    )


def _butterfly(v, op, iota):
    for s in (8, 4, 2, 1):
        v = op(v, _shuffle(v, iota ^ s))
    return v


def _margin_body(logits_hbm, labels_hbm, out_hbm, row_buf, lab_buf, out_buf):
    cid = lax.axis_index("c")
    sid = lax.axis_index("s")
    wid = sid * NUM_CORES + cid
    base = wid * ROWS_PER_WORKER

    pltpu.sync_copy(labels_hbm.at[pl.ds(base, ROWS_PER_WORKER)], lab_buf)

    iota = lax.iota(jnp.int32, LANES)

    def row_step(rl, out_carry):
        out0, out1 = out_carry
        r = base + rl
        pltpu.sync_copy(logits_hbm.at[r], row_buf)
        lblk = (rl // LANES) * LANES
        labv = lab_buf[pl.ds(lblk, LANES)].astype(jnp.float32)
        label_fv = _butterfly(
            jnp.where(iota == rl - lblk, labv, jnp.float32(-1.0)),
            jnp.maximum,
            iota,
        )
        label_i = label_fv[0].astype(jnp.int32)

        def scan_step(j, carry):
            m1, m2, argj = carry
            v = row_buf[pl.ds(j * LANES, LANES)]
            gt = v > m1
            m2 = jnp.maximum(m2, jnp.where(gt, m1, v))
            m1 = jnp.where(gt, v, m1)
            argj = jnp.where(gt, j, argj)
            return m1, m2, argj

        init = (
            jnp.full((LANES,), NEG_INF, jnp.float32),
            jnp.full((LANES,), NEG_INF, jnp.float32),
            jnp.zeros((LANES,), jnp.int32),
        )
        m1, m2, argj = lax.fori_loop(0, NVREG, scan_step, init, unroll=10)

        row_topv = _butterfly(m1, jnp.maximum, iota)
        argcol_vec = (argj * LANES + iota).astype(jnp.float32)
        eq = m1 == row_topv
        argcolv = _butterfly(
            jnp.where(eq, argcol_vec, jnp.float32(2.0**30)), jnp.minimum, iota
        )
        argcol_i = argcolv[0].astype(jnp.int32)
        m1_excl = jnp.where(iota == argcol_i % LANES, NEG_INF, m1)
        row_secondv = _butterfly(jnp.maximum(m1_excl, m2), jnp.maximum, iota)

        cblk = (label_i // LANES) * LANES
        cv = row_buf[pl.ds(cblk, LANES)]
        correctv = _butterfly(
            jnp.where(iota == label_i - cblk, cv, NEG_INF), jnp.maximum, iota
        )
        max_incorrect = jnp.where(argcolv == label_fv, row_secondv, row_topv)
        valv = jnp.minimum(max_incorrect - correctv, KAPPA)

        out0 = jnp.where(iota == rl, valv, out0)
        out1 = jnp.where(iota == rl - LANES, valv, out1)
        return out0, out1

    zeros = jnp.zeros((LANES,), jnp.float32)
    out0, out1 = lax.fori_loop(0, ROWS_PER_WORKER, row_step, (zeros, zeros))
    out_buf[pl.ds(0, LANES)] = out0
    out_buf[pl.ds(LANES, LANES)] = out1
    pltpu.sync_copy(out_buf, out_hbm.at[pl.ds(base, ROWS_PER_WORKER)])


@jax.jit
def _margin_loss(logits, labels):
    mesh = plsc.VectorSubcoreMesh(core_axis_name="c", subcore_axis_name="s")
    fn = functools.partial(
        pl.kernel,
        mesh=mesh,
        out_type=jax.ShapeDtypeStruct((ROWS,), jnp.float32),
        scratch_types=[
            pltpu.VMEM((COLS,), jnp.float32),
            pltpu.VMEM((ROWS_PER_WORKER,), jnp.int32),
            pltpu.VMEM((ROWS_PER_WORKER,), jnp.float32),
        ],
    )(_margin_body)
    return fn(logits, labels)


def kernel(logits, labels):
    return _margin_loss(logits, labels.astype(jnp.int32))
